# baseline (speedup 1.0000x reference)
import functools

import jax
import jax.numpy as jnp
from jax import lax
from jax.experimental import pallas as pl
from jax.experimental.pallas import tpu as pltpu
from jax.experimental.pallas import tpu_sc as plsc

_N_WORKERS = 32


def _detile_kernel(v, d):
    nblk = v // 128
    tail = v - nblk * 128
    mesh = plsc.VectorSubcoreMesh(core_axis_name="c", subcore_axis_name="s")
    per_w = nblk // _N_WORKERS
    rem = nblk - per_w * _N_WORKERS

    @functools.partial(
        pl.kernel,
        mesh=mesh,
        out_type=jax.ShapeDtypeStruct((v * d,), jnp.float32),
        scratch_types=[
            pltpu.VMEM((d, 128), jnp.float32),
            pltpu.VMEM((d, 128), jnp.float32),
            pltpu.VMEM((128 * d,), jnp.float32),
            pltpu.VMEM((128 * d,), jnp.float32),
            pltpu.SemaphoreType.DMA,
            pltpu.SemaphoreType.DMA,
            pltpu.SemaphoreType.DMA,
            pltpu.SemaphoreType.DMA,
        ],
        compiler_params=pltpu.CompilerParams(
            use_tc_tiling_on_sc=True, needs_layout_passes=False
        ),
    )
    def k(wt_hbm, tail_hbm, out_hbm, blk0, blk1, rows0, rows1, gs0, gs1, ws0, ws1):
        blks = (blk0, blk1)
        rows = (rows0, rows1)
        wid = lax.axis_index("s") * 2 + lax.axis_index("c")
        lo = wid * per_w + jnp.minimum(wid, rem)
        n_w = per_w + jnp.where(wid < rem, 1, 0)
        gsems = (gs0, gs1)
        wsems = (ws0, ws1)

        lanes = lax.iota(jnp.int32, 16)

        def start_in(i, b):
            return pltpu.async_copy(
                wt_hbm.at[:, pl.ds(128 * (lo + i), 128)], blks[b], gsems[b]
            )

        def transpose_block(src, dst):
            @plsc.parallel_loop(0, 128, unroll=16)
            def _row(t):
                t_vec = jnp.full((16,), t, jnp.int32)
                for j in range(d // 16):
                    val = plsc.load_gather(src, [16 * j + lanes, t_vec])
                    dst[pl.ds(d * t + 16 * j, 16)] = val

        def wait_in(b):
            pltpu.make_async_copy(
                wt_hbm.at[:, pl.ds(0, 128)], blks[b], gsems[b]
            ).wait()

        def wait_out(b):
            pltpu.make_async_copy(
                rows[b], out_hbm.at[pl.ds(0, 128 * d)], wsems[b]
            ).wait()

        def step(i, b, prefetch):
            @pl.when(prefetch[0] < n_w)
            def _():
                start_in(*prefetch)

            wait_in(b)

            @pl.when(i >= 2)
            def _():
                wait_out(b)

            transpose_block(blks[b], rows[b])
            pltpu.async_copy(
                rows[b],
                out_hbm.at[pl.ds(128 * d * (lo + i), 128 * d)],
                wsems[b],
            )

        start_in(0, 0)

        def pair_body(p, carry):
            i0 = 2 * p
            i1 = i0 + 1
            step(i0, 0, (i1, 1))

            @pl.when(i1 < n_w)
            def _():
                step(i1, 1, (i1 + 1, 0))

            return carry

        lax.fori_loop(0, (n_w + 1) // 2, pair_body, 0)

        for b in range(2):
            wait_out(b)

        if tail:
            @pl.when(wid == _N_WORKERS - 1)
            def _():
                pltpu.sync_copy(tail_hbm, rows0.at[pl.ds(0, tail * d)])
                pltpu.sync_copy(
                    rows0.at[pl.ds(0, tail * d)],
                    out_hbm.at[pl.ds(128 * d * nblk, tail * d)],
                )

    return k


def _gather_kernel(n_tokens, dim, chunk, nbuf, dist):
    per_w = n_tokens // _N_WORKERS
    n_chunks = per_w // chunk
    mesh = plsc.VectorSubcoreMesh(core_axis_name="c", subcore_axis_name="s")

    @functools.partial(
        pl.kernel,
        mesh=mesh,
        out_type=jax.ShapeDtypeStruct((n_tokens, dim), jnp.float32),
        scratch_types=(
            [
                pltpu.VMEM((per_w,), jnp.int32),
                pltpu.VMEM((nbuf, chunk, dim), jnp.float32),
            ]
            + [pltpu.SemaphoreType.DMA] * (2 * nbuf)
        ),
        compiler_params=pltpu.CompilerParams(use_tc_tiling_on_sc=False),
    )
    def k(idx_hbm, table_hbm, out_hbm, idx_v, rows_v, *sems):
        gsems = sems[:nbuf]
        wsems = sems[nbuf:]
        wid = lax.axis_index("s") * 2 + lax.axis_index("c")
        base = wid * per_w

        pltpu.sync_copy(idx_hbm.at[pl.ds(base, per_w)], idx_v)

        pending_g = {}
        pending_w = {}

        def start_gather(j):
            b = j % nbuf
            pending_g[b] = pltpu.async_copy(
                table_hbm.at[idx_v.at[pl.ds(j * chunk, chunk)]],
                rows_v.at[b],
                gsems[b],
            )

        for j in range(min(dist, n_chunks)):
            start_gather(j)
        for i in range(n_chunks):
            b = i % nbuf
            pending_g.pop(b).wait()
            pending_w[b] = pltpu.async_copy(
                rows_v.at[b], out_hbm.at[pl.ds(base + i * chunk, chunk)], wsems[b]
            )
            j = i + dist
            if j < n_chunks:
                bj = j % nbuf
                if bj in pending_w:
                    pending_w.pop(bj).wait()
                start_gather(j)
        for w in pending_w.values():
            w.wait()

    return k


def kernel(token_ids, weight):
    b, s = token_ids.shape
    v, d = weight.shape
    n = b * s
    chunk = 512
    assert (n // _N_WORKERS) % chunk == 0
    flat = token_ids.reshape(n).astype(jnp.int32)
    nblk = v // 128
    tail_lin = weight[128 * nblk:].reshape(-1)
    table_lin = _detile_kernel(v, d)(weight.T, tail_lin).reshape(v, d)
    out = _gather_kernel(n, d, chunk, nbuf=3, dist=2)(flat, table_lin)
    return out.reshape(b, s, d)


# speedup vs baseline: 1.2009x; 1.2009x over previous
import functools

import jax
import jax.numpy as jnp
from jax import lax
from jax.experimental import pallas as pl
from jax.experimental.pallas import tpu as pltpu
from jax.experimental.pallas import tpu_sc as plsc

_N_WORKERS = 32


def _gather_kernel(n_tokens, dim, chunk, nbuf, dist):
    per_w = n_tokens // _N_WORKERS
    n_chunks = per_w // chunk
    mesh = plsc.VectorSubcoreMesh(core_axis_name="c", subcore_axis_name="s")

    @functools.partial(
        pl.kernel,
        mesh=mesh,
        out_type=jax.ShapeDtypeStruct((n_tokens, dim), jnp.float32),
        scratch_types=(
            [
                pltpu.VMEM((per_w,), jnp.int32),
                pltpu.VMEM((nbuf, chunk, dim), jnp.float32),
            ]
            + [pltpu.SemaphoreType.DMA] * (2 * nbuf)
        ),
        compiler_params=pltpu.CompilerParams(use_tc_tiling_on_sc=False),
    )
    def k(idx_hbm, table_hbm, out_hbm, idx_v, rows_v, *sems):
        gsems = sems[:nbuf]
        wsems = sems[nbuf:]
        wid = lax.axis_index("s") * 2 + lax.axis_index("c")
        base = wid * per_w

        pltpu.sync_copy(idx_hbm.at[pl.ds(base, per_w)], idx_v)

        pending_g = {}
        pending_w = {}

        def start_gather(j):
            b = j % nbuf
            pending_g[b] = pltpu.async_copy(
                table_hbm.at[idx_v.at[pl.ds(j * chunk, chunk)]],
                rows_v.at[b],
                gsems[b],
            )

        for j in range(min(dist, n_chunks)):
            start_gather(j)
        for i in range(n_chunks):
            b = i % nbuf
            pending_g.pop(b).wait()
            pending_w[b] = pltpu.async_copy(
                rows_v.at[b], out_hbm.at[pl.ds(base + i * chunk, chunk)], wsems[b]
            )
            j = i + dist
            if j < n_chunks:
                bj = j % nbuf
                if bj in pending_w:
                    pending_w.pop(bj).wait()
                start_gather(j)
        for w in pending_w.values():
            w.wait()

    return k


def kernel(token_ids, weight):
    b, s = token_ids.shape
    v, d = weight.shape
    n = b * s
    chunk = 512
    assert (n // _N_WORKERS) % chunk == 0
    flat = token_ids.reshape(n).astype(jnp.int32)
    out = _gather_kernel(n, d, chunk, nbuf=3, dist=2)(flat, weight)
    return out.reshape(b, s, d)


# speedup vs baseline: 1.2040x; 1.0026x over previous
import functools

import jax
import jax.numpy as jnp
from jax import lax
from jax.experimental import pallas as pl
from jax.experimental.pallas import tpu as pltpu
from jax.experimental.pallas import tpu_sc as plsc

_N_WORKERS = 32


def _gather_kernel(n_tokens, dim, chunk, nbuf, dist):
    per_w = n_tokens // _N_WORKERS
    n_chunks = per_w // chunk
    mesh = plsc.VectorSubcoreMesh(core_axis_name="c", subcore_axis_name="s")

    @functools.partial(
        pl.kernel,
        mesh=mesh,
        out_type=jax.ShapeDtypeStruct((n_tokens, dim), jnp.float32),
        scratch_types=(
            [
                pltpu.VMEM((per_w,), jnp.int32),
                pltpu.VMEM((nbuf, chunk, dim), jnp.float32),
            ]
            + [pltpu.SemaphoreType.DMA] * (2 * nbuf)
        ),
        compiler_params=pltpu.CompilerParams(use_tc_tiling_on_sc=False),
    )
    def k(idx_hbm, table_hbm, out_hbm, idx_v, rows_v, *sems):
        gsems = sems[:nbuf]
        wsems = sems[nbuf:]
        wid = lax.axis_index("s") * 2 + lax.axis_index("c")
        base = wid * per_w

        pltpu.sync_copy(idx_hbm.at[pl.ds(base, per_w)], idx_v)

        pending_g = {}
        pending_w = {}

        def start_gather(j):
            b = j % nbuf
            pending_g[b] = pltpu.async_copy(
                table_hbm.at[idx_v.at[pl.ds(j * chunk, chunk)]],
                rows_v.at[b],
                gsems[b],
            )

        for j in range(min(dist, n_chunks)):
            start_gather(j)
        for i in range(n_chunks):
            b = i % nbuf
            pending_g.pop(b).wait()
            pending_w[b] = pltpu.async_copy(
                rows_v.at[b], out_hbm.at[pl.ds(base + i * chunk, chunk)], wsems[b]
            )
            j = i + dist
            if j < n_chunks:
                bj = j % nbuf
                if bj in pending_w:
                    pending_w.pop(bj).wait()
                start_gather(j)
        for w in pending_w.values():
            w.wait()

    return k


def kernel(token_ids, weight):
    b, s = token_ids.shape
    v, d = weight.shape
    n = b * s
    chunk = 416
    assert (n // _N_WORKERS) % chunk == 0
    flat = token_ids.reshape(n).astype(jnp.int32)
    out = _gather_kernel(n, d, chunk, nbuf=4, dist=3)(flat, weight)
    return out.reshape(b, s, d)
